# baseline (speedup 1.0000x reference)
import functools

import jax
import jax.numpy as jnp
from jax import lax
from jax.experimental import pallas as pl
from jax.experimental.pallas import tpu as pltpu
from jax.experimental.pallas import tpu_sc as plsc

B, T, D_MODEL = 4, 8192, 768
N_EXPERTS = 64
TOP_K = 2
N_TOKENS = B * T
TOKEN_BLOCK = 4096
BLOCKS_PER_BATCH = T // TOKEN_BLOCK
N_BLOCKS = N_TOKENS // TOKEN_BLOCK


def _router_kernel(x_ref, wt_ref, idx_ref, scores_ref, probs_ref, imp_ref, load_ref):
    i = pl.program_id(0)

    @pl.when(i == 0)
    def _init():
        imp_ref[...] = jnp.zeros_like(imp_ref)
        load_ref[...] = jnp.zeros_like(load_ref)

    xb = x_ref[0]
    wt = wt_ref[...]
    logits = jax.lax.dot_general(
        wt, xb, (((1,), (1,)), ((), ())), preferred_element_type=jnp.float32
    )

    iota = jax.lax.broadcasted_iota(jnp.int32, logits.shape, 0)

    m1 = jnp.max(logits, axis=0, keepdims=True)
    is1 = logits == m1
    i1 = jnp.min(jnp.where(is1, iota, N_EXPERTS), axis=0, keepdims=True)
    oh1 = iota == i1

    masked = jnp.where(oh1, -jnp.inf, logits)
    m2 = jnp.max(masked, axis=0, keepdims=True)
    is2 = masked == m2
    i2 = jnp.min(jnp.where(is2, iota, N_EXPERTS), axis=0, keepdims=True)
    oh2 = iota == i2

    ex = jnp.exp(logits - m1)
    denom = jnp.sum(ex, axis=0, keepdims=True)
    probs = ex / denom
    probs_ref[0] = probs

    e2 = jnp.exp(m2 - m1)
    s1 = 1.0 / (1.0 + e2)
    s2 = 1.0 - s1
    scores_ref[0] = jnp.concatenate([s1, s2], axis=0)
    idx_ref[0] = jnp.concatenate([i1, i2], axis=0)

    imp_ref[...] += jnp.sum(probs, axis=1)
    load_ref[...] += jnp.sum(oh1.astype(jnp.float32) + oh2.astype(jnp.float32), axis=1)

    @pl.when(i == N_BLOCKS - 1)
    def _finish():
        imp_ref[...] = imp_ref[...] * (1.0 / N_TOKENS)
        load_ref[...] = load_ref[...] * (1.0 / (N_TOKENS * TOP_K))


_NC = 2
_NS = 16
_L = 16
_NW = _NC * _NS
_TOTAL_IDX = N_TOKENS * TOP_K
_CHUNK = _TOTAL_IDX // _NW
_GROUP = 512
_N_GROUPS = _CHUNK // _GROUP


_ROW = 128


def _sc_histogram_kernel(
    idx_hbm, ones_hbm, zeros_hbm, midx_hbm, out_hbm, idx_v, ones_v, acc_v, shared, shared2
):
    cid = lax.axis_index("c")
    sid = lax.axis_index("s")
    wid = sid * _NC + cid

    pltpu.sync_copy(zeros_hbm, acc_v)
    pltpu.sync_copy(acc_v, shared.at[pl.ds(sid * N_EXPERTS, N_EXPERTS)])

    @pl.when(sid == 0)
    def _init2():
        pltpu.sync_copy(acc_v, shared2)

    pltpu.sync_copy(ones_hbm, ones_v)
    plsc.subcore_barrier()

    for g in range(_N_GROUPS):
        pltpu.sync_copy(
            idx_hbm.at[pl.ds(wid * _CHUNK + g * _GROUP, _GROUP)], idx_v
        )
        for c in range(_GROUP // _L):
            sl = pl.ds(c * _L, _L)
            idx_v[sl] = idx_v[sl] + sid * N_EXPERTS
        pltpu.sync_copy(ones_v, shared.at[idx_v], add=True)
    plsc.subcore_barrier()

    @pl.when(sid == 0)
    def _merge():
        for h in range(_NS * N_EXPERTS // _GROUP):
            pltpu.sync_copy(shared.at[pl.ds(h * _GROUP, _GROUP)], ones_v)
            pltpu.sync_copy(midx_hbm.at[pl.ds(h * _GROUP, _GROUP)], idx_v)
            pltpu.sync_copy(ones_v, shared2.at[idx_v], add=True)
        pltpu.sync_copy(shared2, acc_v)
        pltpu.sync_copy(acc_v, out_hbm.at[cid])


@functools.partial(
    pl.kernel,
    mesh=plsc.VectorSubcoreMesh(core_axis_name="c", subcore_axis_name="s"),
    out_type=jax.ShapeDtypeStruct((_NC, N_EXPERTS, _ROW), jnp.float32),
    scratch_types=[
        pltpu.VMEM((_GROUP,), jnp.int32),
        pltpu.VMEM((_GROUP, _ROW), jnp.float32),
        pltpu.VMEM((N_EXPERTS, _ROW), jnp.float32),
        pltpu.VMEM_SHARED((_NS * N_EXPERTS, _ROW), jnp.float32),
        pltpu.VMEM_SHARED((N_EXPERTS, _ROW), jnp.float32),
    ],
)
def _sc_histogram(
    idx_hbm, ones_hbm, zeros_hbm, midx_hbm, out_hbm, idx_v, ones_v, acc_v, shared, shared2
):
    _sc_histogram_kernel(
        idx_hbm, ones_hbm, zeros_hbm, midx_hbm, out_hbm, idx_v, ones_v, acc_v, shared, shared2
    )


@jax.jit
def kernel(x, W_gate):
    out_shapes = (
        jax.ShapeDtypeStruct((B, TOP_K, T), jnp.int32),
        jax.ShapeDtypeStruct((B, TOP_K, T), jnp.float32),
        jax.ShapeDtypeStruct((B, N_EXPERTS, T), jnp.float32),
        jax.ShapeDtypeStruct((N_EXPERTS,), jnp.float32),
        jax.ShapeDtypeStruct((N_EXPERTS,), jnp.float32),
    )
    idx_t, scores_t, probs_t, imp, load = pl.pallas_call(
        _router_kernel,
        grid=(N_BLOCKS,),
        in_specs=[
            pl.BlockSpec(
                (1, TOKEN_BLOCK, D_MODEL),
                lambda i: (i // BLOCKS_PER_BATCH, i % BLOCKS_PER_BATCH, 0),
            ),
            pl.BlockSpec((N_EXPERTS, D_MODEL), lambda i: (0, 0)),
        ],
        out_specs=(
            pl.BlockSpec(
                (1, TOP_K, TOKEN_BLOCK),
                lambda i: (i // BLOCKS_PER_BATCH, 0, i % BLOCKS_PER_BATCH),
            ),
            pl.BlockSpec(
                (1, TOP_K, TOKEN_BLOCK),
                lambda i: (i // BLOCKS_PER_BATCH, 0, i % BLOCKS_PER_BATCH),
            ),
            pl.BlockSpec(
                (1, N_EXPERTS, TOKEN_BLOCK),
                lambda i: (i // BLOCKS_PER_BATCH, 0, i % BLOCKS_PER_BATCH),
            ),
            pl.BlockSpec((N_EXPERTS,), lambda i: (0,)),
            pl.BlockSpec((N_EXPERTS,), lambda i: (0,)),
        ),
        out_shape=out_shapes,
    )(x, W_gate.T)

    hist = _sc_histogram(
        idx_t.reshape(_TOTAL_IDX),
        jnp.ones((_GROUP, _ROW), jnp.float32),
        jnp.zeros((N_EXPERTS, _ROW), jnp.float32),
        jnp.tile(jnp.arange(N_EXPERTS, dtype=jnp.int32), _NS),
    )
    load_sc = (hist[0, :, 0] + hist[1, :, 0]) * (1.0 / _TOTAL_IDX)

    idx = jnp.swapaxes(idx_t, 1, 2)
    scores = jnp.swapaxes(scores_t, 1, 2)
    probs = jnp.swapaxes(probs_t, 1, 2)
    del load
    return (idx, scores, probs, imp, load_sc)


# speedup vs baseline: 2.4439x; 2.4439x over previous
import jax
import jax.numpy as jnp
from jax.experimental import pallas as pl

B, T, D_MODEL = 4, 8192, 768
N_EXPERTS = 64
TOP_K = 2
N_TOKENS = B * T
TOKEN_BLOCK = 4096
BLOCKS_PER_BATCH = T // TOKEN_BLOCK
N_BLOCKS = N_TOKENS // TOKEN_BLOCK


def _router_kernel(x_ref, wt_ref, idx_ref, scores_ref, probs_ref, imp_ref, load_ref):
    i = pl.program_id(0)

    @pl.when(i == 0)
    def _init():
        imp_ref[...] = jnp.zeros_like(imp_ref)
        load_ref[...] = jnp.zeros_like(load_ref)

    xb = x_ref[0]
    wt = wt_ref[...]
    logits = jax.lax.dot_general(
        wt, xb, (((1,), (1,)), ((), ())), preferred_element_type=jnp.float32
    )

    iota = jax.lax.broadcasted_iota(jnp.int32, logits.shape, 0)

    m1 = jnp.max(logits, axis=0, keepdims=True)
    is1 = logits == m1
    i1 = jnp.min(jnp.where(is1, iota, N_EXPERTS), axis=0, keepdims=True)
    oh1 = iota == i1

    masked = jnp.where(oh1, -jnp.inf, logits)
    m2 = jnp.max(masked, axis=0, keepdims=True)
    is2 = masked == m2
    i2 = jnp.min(jnp.where(is2, iota, N_EXPERTS), axis=0, keepdims=True)
    oh2 = iota == i2

    ex = jnp.exp(logits - m1)
    denom = jnp.sum(ex, axis=0, keepdims=True)
    probs = ex * (1.0 / denom)
    probs_ref[0] = probs

    e2 = jnp.exp(m2 - m1)
    s1 = 1.0 / (1.0 + e2)
    s2 = 1.0 - s1
    scores_ref[0] = jnp.concatenate([s1, s2], axis=0)
    idx_ref[0] = jnp.concatenate([i1, i2], axis=0)

    imp_ref[...] += jnp.sum(probs, axis=1)
    load_ref[...] += jnp.sum(oh1.astype(jnp.float32) + oh2.astype(jnp.float32), axis=1)

    @pl.when(i == N_BLOCKS - 1)
    def _finish():
        imp_ref[...] = imp_ref[...] * (1.0 / N_TOKENS)
        load_ref[...] = load_ref[...] * (1.0 / (N_TOKENS * TOP_K))


@jax.jit
def kernel(x, W_gate):
    out_shapes = (
        jax.ShapeDtypeStruct((B, TOP_K, T), jnp.int32),
        jax.ShapeDtypeStruct((B, TOP_K, T), jnp.float32),
        jax.ShapeDtypeStruct((B, N_EXPERTS, T), jnp.float32),
        jax.ShapeDtypeStruct((N_EXPERTS,), jnp.float32),
        jax.ShapeDtypeStruct((N_EXPERTS,), jnp.float32),
    )
    idx_t, scores_t, probs_t, imp, load = pl.pallas_call(
        _router_kernel,
        grid=(N_BLOCKS,),
        in_specs=[
            pl.BlockSpec(
                (1, TOKEN_BLOCK, D_MODEL),
                lambda i: (i // BLOCKS_PER_BATCH, i % BLOCKS_PER_BATCH, 0),
            ),
            pl.BlockSpec((N_EXPERTS, D_MODEL), lambda i: (0, 0)),
        ],
        out_specs=(
            pl.BlockSpec(
                (1, TOP_K, TOKEN_BLOCK),
                lambda i: (i // BLOCKS_PER_BATCH, 0, i % BLOCKS_PER_BATCH),
            ),
            pl.BlockSpec(
                (1, TOP_K, TOKEN_BLOCK),
                lambda i: (i // BLOCKS_PER_BATCH, 0, i % BLOCKS_PER_BATCH),
            ),
            pl.BlockSpec(
                (1, N_EXPERTS, TOKEN_BLOCK),
                lambda i: (i // BLOCKS_PER_BATCH, 0, i % BLOCKS_PER_BATCH),
            ),
            pl.BlockSpec((N_EXPERTS,), lambda i: (0,)),
            pl.BlockSpec((N_EXPERTS,), lambda i: (0,)),
        ),
        out_shape=out_shapes,
    )(x, W_gate.T)

    idx = jnp.swapaxes(idx_t, 1, 2)
    scores = jnp.swapaxes(scores_t, 1, 2)
    probs = jnp.swapaxes(probs_t, 1, 2)
    return (idx, scores, probs, imp, load)
